# baseline (speedup 1.0000x reference)
import numpy as np
import jax
import jax.numpy as jnp
from jax.experimental import pallas as pl
from jax.experimental.pallas import tpu as pltpu
from jax._src.random.threefry2x32 import threefry2x32_p

_E_CRIT_NEW = 26267.160775850585
_F_STD = 0.001
_R_STD0 = 0.01
_R_INIT = 0.08076263685971334
_G_STD = 0.01
_NU = 1.0 / (_G_STD * np.sqrt(2.0 * np.pi))
_TINY = np.float32(np.finfo(np.float32).tiny)


def _voc(soc):
    v_L = -1.59614486
    v_0 = 4.13646328
    gamma = 0.63726463
    alpha = 1.40174122
    beta = 2.54478965
    return (v_L + (v_0 - v_L) * jnp.exp(gamma * (soc - 1.0))
            + alpha * v_L * (soc - 1.0)
            + (1.0 - alpha) * v_L * (np.exp(-beta) - jnp.exp(-beta * jnp.sqrt(soc))))


def _gumbel_bits(k1, k2, c2):
    zero = jnp.zeros_like(c2)
    b1, b2 = threefry2x32_p.bind(k1, k2, zero, c2)
    bits = b1 ^ b2
    fb = (bits >> jnp.uint32(9)) | jnp.uint32(0x3F800000)
    f = jax.lax.bitcast_convert_type(fb, jnp.float32) - jnp.float32(1.0)
    u = jnp.maximum(_TINY, f + _TINY)
    return -jnp.log(-jnp.log(u))


def _propagate(t, soc, R, noise_r, noise_s, curr_ref, vm_ref, rstd_ref, cst_ref):
    i_old = curr_ref[jnp.maximum(t - 1, 0)]
    R = R + rstd_ref[t] * noise_r
    V = _voc(soc) - i_old * R
    soc = soc - i_old * V / np.float32(_E_CRIT_NEW) * cst_ref[0]
    soc = soc + np.float32(_F_STD) * noise_s
    soc = jnp.where(soc > 1.0, 1.0, soc)
    soc = jnp.where(soc < 0.0, 1e-10, soc)
    i_new = curr_ref[t]
    V = _voc(soc) - i_new * R
    q = (V - vm_ref[t]) / np.float32(_G_STD)
    logW = cst_ref[1] - 0.5 * (q * q)
    m = jnp.max(logW)
    logits = logW - m
    step_loss = m + jnp.log(jnp.sum(jnp.exp(logits))) - cst_ref[2]
    return soc, R, V, logits, step_loss


def _make_kernel(N, T, RT, P, CH):
    NC = N // CH
    dot_dims = (((1,), (1,)), ((), ()))

    def body(keys_ref, curr_ref, vm_ref, rstd_ref, cst_ref,
             soc0_ref, rn_ref, sn_ref,
             v_ref, s_ref, loss_ref,
             soc_r, R_r, V_r, lg_r, nsoc_r, nR_r, nV_r, acc_ref):
        t = pl.program_id(0)
        p = pl.program_id(1)
        k1 = keys_ref[t, 0]
        k2 = keys_ref[t, 1]

        @pl.when(p == 0)
        def _prop_phase():
            soc = jnp.where(t == 0, soc0_ref[...], nsoc_r[...])
            R = jnp.where(t == 0, jnp.float32(_R_INIT), nR_r[...])
            soc, R, V, logits, sl = _propagate(
                t, soc, R, rn_ref[0], sn_ref[0],
                curr_ref, vm_ref, rstd_ref, cst_ref)
            soc_r[...] = soc
            R_r[...] = R
            V_r[...] = V
            lg_r[...] = logits
            acc_ref[0, 0] = jnp.where(t == 0, sl, acc_ref[0, 0] + sl)

        def _arg_body(c, carry):
            run_max, run_idx = carry
            il = jax.lax.broadcasted_iota(jnp.int32, (RT, CH), 0) + p * RT
            jr = jax.lax.broadcasted_iota(jnp.int32, (RT, CH), 1) + c * CH
            c2 = (il * N + jr).astype(jnp.uint32)
            g = _gumbel_bits(k1, k2, c2)
            val = g + lg_r[:, pl.ds(c * CH, CH)]
            rm = jnp.max(val, axis=1, keepdims=True)
            ri = jnp.min(jnp.where(val == rm, jr, N), axis=1, keepdims=True)
            better = rm > run_max
            return (jnp.maximum(run_max, rm),
                    jnp.where(better, ri, run_idx))

        neg_inf = jnp.float32(np.float32(-np.inf))
        _, idx = jax.lax.fori_loop(
            0, NC, _arg_body,
            (jnp.full((RT, 1), neg_inf, jnp.float32),
             jnp.full((RT, 1), N, jnp.int32)))

        def _gather_body(c, accs):
            asoc, aR, aV = accs
            jr = jax.lax.broadcasted_iota(jnp.int32, (RT, CH), 1) + c * CH
            oh = (jr == idx).astype(jnp.float32)
            sl = pl.ds(c * CH, CH)
            hi = jax.lax.Precision.HIGHEST
            asoc = asoc + jax.lax.dot_general(
                soc_r[:, sl], oh, dot_dims, precision=hi,
                preferred_element_type=jnp.float32)
            aR = aR + jax.lax.dot_general(
                R_r[:, sl], oh, dot_dims, precision=hi,
                preferred_element_type=jnp.float32)
            aV = aV + jax.lax.dot_general(
                V_r[:, sl], oh, dot_dims, precision=hi,
                preferred_element_type=jnp.float32)
            return asoc, aR, aV

        zero = jnp.zeros((1, RT), jnp.float32)
        nsoc, nR, nV = jax.lax.fori_loop(0, NC, _gather_body, (zero, zero, zero))
        ds = pl.ds(p * RT, RT)
        nsoc_r[:, ds] = nsoc
        nR_r[:, ds] = nR
        nV_r[:, ds] = nV
        v_ref[0, :, ds] = nV
        s_ref[0, :, ds] = nsoc

        @pl.when(jnp.logical_and(t == T - 1, p == P - 1))
        def _final():
            loss_ref[0, 0] = acc_ref[0, 0]

    return body


def kernel(soc_init, current, voltage_measured, E_crit):
    N = soc_init.shape[0]
    T = current.shape[1]
    RT = 128 if N % 128 == 0 else 8
    P = N // RT
    CH = 2048 if N % 2048 == 0 else N

    nkey = jax.random.key(42)
    ts = jnp.arange(T)
    keys_R = jax.vmap(lambda i: jax.random.fold_in(nkey, 3 * i))(ts)
    keys_s = jax.vmap(lambda i: jax.random.fold_in(nkey, 3 * i + 1))(ts)
    keys_r = jax.vmap(lambda i: jax.random.fold_in(nkey, 3 * i + 2))(ts)
    rnoise = jax.vmap(lambda k: jax.random.normal(k, (N, 1), jnp.float32))(keys_R)
    snoise = jax.vmap(lambda k: jax.random.normal(k, (N, 1), jnp.float32))(keys_s)
    kr_data = jax.vmap(jax.random.key_data)(keys_r).astype(jnp.uint32)

    rstd = jnp.asarray(_R_STD0 * np.exp(-np.arange(T) / 100.0), dtype=jnp.float32)
    log_nu = jnp.log(jnp.float32(_NU))
    log_N = jnp.float32(np.log(N))
    cst = jnp.stack([E_crit.astype(jnp.float32)[0], log_nu, log_N])

    rn = rnoise.reshape(T, 1, N)
    sn = snoise.reshape(T, 1, N)
    soc0 = soc_init.astype(jnp.float32).reshape(1, N)

    f32 = jnp.float32
    grid_spec = pltpu.PrefetchScalarGridSpec(
        num_scalar_prefetch=5,
        grid=(T, P),
        in_specs=[
            pl.BlockSpec((1, N), lambda t, p, *_: (0, 0)),
            pl.BlockSpec((1, 1, N), lambda t, p, *_: (t, 0, 0)),
            pl.BlockSpec((1, 1, N), lambda t, p, *_: (t, 0, 0)),
        ],
        out_specs=[
            pl.BlockSpec((1, 1, N), lambda t, p, *_: (t, 0, 0)),
            pl.BlockSpec((1, 1, N), lambda t, p, *_: (t, 0, 0)),
            pl.BlockSpec(memory_space=pltpu.SMEM),
        ],
        scratch_shapes=[
            pltpu.VMEM((1, N), f32), pltpu.VMEM((1, N), f32),
            pltpu.VMEM((1, N), f32), pltpu.VMEM((1, N), f32),
            pltpu.VMEM((1, N), f32), pltpu.VMEM((1, N), f32),
            pltpu.VMEM((1, N), f32),
            pltpu.SMEM((1, 1), f32),
        ],
    )
    out_shape = [
        jax.ShapeDtypeStruct((T, 1, N), f32),
        jax.ShapeDtypeStruct((T, 1, N), f32),
        jax.ShapeDtypeStruct((1, 1), f32),
    ]
    v_out, s_out, loss = pl.pallas_call(
        _make_kernel(N, T, RT, P, CH),
        grid_spec=grid_spec,
        out_shape=out_shape,
        compiler_params=pltpu.CompilerParams(
            dimension_semantics=("arbitrary", "arbitrary")),
    )(kr_data, current[0], voltage_measured[0], rstd, cst, soc0, rn, sn)

    voltage = v_out[:, 0, :].T
    soc_hist = s_out[:, 0, :].T
    return (loss.reshape(1), voltage, soc_hist)
